# baseline (speedup 1.0000x reference)
import functools

import jax
import jax.numpy as jnp
from jax import lax
from jax.experimental import pallas as pl
from jax.experimental.pallas import tpu as pltpu
from jax.experimental.pallas import tpu_sc as plsc

D = 128
K = 16
NC = 2
NS = 16
NW = NC * NS
CH = 8
CHK = CH * K
SCH = 64
ZR = 32
NB = 4


def _sc_gather_sum(features, nodes_p, n0_p, n1_p, bp):
    npw = bp // NW
    n_chunks = npw // CH
    s_chunks = npw // SCH
    z_copies = npw // ZR
    mesh = plsc.VectorSubcoreMesh(core_axis_name="c", subcore_axis_name="s")
    f32 = jnp.float32

    @functools.partial(
        pl.kernel,
        out_type=(
            jax.ShapeDtypeStruct((bp, D), f32),
            jax.ShapeDtypeStruct((bp, D), f32),
            jax.ShapeDtypeStruct((bp, D), f32),
        ),
        mesh=mesh,
        scratch_types=[
            pltpu.VMEM((npw,), jnp.int32),
            pltpu.VMEM((npw * K,), jnp.int32),
            pltpu.VMEM((NB, CHK, D), f32),
            pltpu.VMEM((ZR, D), f32),
            pltpu.VMEM((NB, CHK), jnp.int32),
            pltpu.VMEM_SHARED((NS * npw, D), f32),
            pltpu.SemaphoreType.DMA,
            pltpu.SemaphoreType.DMA,
            pltpu.SemaphoreType.DMA,
            pltpu.SemaphoreType.DMA,
            pltpu.SemaphoreType.DMA,
            pltpu.SemaphoreType.DMA,
            pltpu.SemaphoreType.DMA,
            pltpu.SemaphoreType.DMA,
            pltpu.SemaphoreType.DMA,
            pltpu.SemaphoreType.DMA,
        ],
    )
    def sc_kernel(feat_hbm, nodes_hbm, n0_hbm, n1_hbm,
                  s_hbm, n0s_hbm, n1s_hbm,
                  sidx_v, nidx_v, rows_v, zer_v, aidx_v, acc_sh,
                  sg0, sg1, sg2, sg3, sa0, sa1, sa2, sa3, sem_s, sem_o):
        sem_g = (sg0, sg1, sg2, sg3)
        sem_a = (sa0, sa1, sa2, sa3)
        sid = lax.axis_index("s")
        wid = sid * NC + lax.axis_index("c")
        nbase = wid * npw
        abase = sid * npw

        for r in range(ZR):
            for j in range(D // 16):
                zer_v[r, pl.ds(j * 16, 16)] = jnp.zeros((16,), f32)

        def zero_slab():
            for zi in range(z_copies):
                pltpu.sync_copy(zer_v, acc_sh.at[pl.ds(abase + zi * ZR, ZR)])

        zero_slab()

        pltpu.sync_copy(nodes_hbm.at[pl.ds(nbase, npw)], sidx_v)

        def one_table(tbl_hbm, out_hbm):
            ibase = nbase * K
            pltpu.sync_copy(tbl_hbm.at[pl.ds(ibase, npw * K)], nidx_v)

            def issue_gather(ci, b):
                pltpu.async_copy(
                    feat_hbm.at[nidx_v.at[pl.ds(ci * CHK, CHK)]],
                    rows_v.at[b], sem_g[b])

            def wait_gather(b):
                pltpu.make_async_copy(
                    feat_hbm.at[nidx_v.at[pl.ds(0, CHK)]],
                    rows_v.at[b], sem_g[b]).wait()

            def issue_add(ci, b):
                for j in range(CH):
                    aidx_v[b, pl.ds(j * K, K)] = lax.full(
                        (16,), abase + ci * CH + j, jnp.int32)
                pltpu.async_copy(rows_v.at[b], acc_sh.at[aidx_v.at[b]],
                                 sem_a[b], add=True)

            def wait_add(b):
                pltpu.make_async_copy(rows_v.at[b], acc_sh.at[aidx_v.at[b]],
                                      sem_a[b]).wait()

            for b in range(NB):
                issue_gather(b, b)

            @pl.loop(0, n_chunks, step=NB)
            def _(g):
                for b in range(NB):
                    ci = g + b
                    pb = (b - 1) % NB
                    pci = ci - 1

                    @pl.when((pci >= 0) & (pci + NB < n_chunks))
                    def _():
                        wait_add(pb)
                        issue_gather(pci + NB, pb)

                    wait_gather(b)
                    issue_add(ci, b)

            for b in range(NB):
                wait_add(b)
            pltpu.async_copy(acc_sh.at[pl.ds(abase, npw)],
                             out_hbm.at[pl.ds(nbase, npw)], sem_o)

        one_table(n0_hbm, n0s_hbm)

        s_parts = []
        off = 0
        while off < npw:
            sz = min(CHK, npw - off)
            s_parts.append((off, sz))
            off += sz
        for b, (soff, sz) in enumerate(s_parts):
            pltpu.async_copy(feat_hbm.at[sidx_v.at[pl.ds(soff, sz)]],
                             rows_v.at[b, pl.ds(0, sz)], sem_g[b])
        for b, (soff, sz) in enumerate(s_parts):
            pltpu.make_async_copy(feat_hbm.at[sidx_v.at[pl.ds(soff, sz)]],
                                  rows_v.at[b, pl.ds(0, sz)], sem_g[b]).wait()
            pltpu.async_copy(rows_v.at[b, pl.ds(0, sz)],
                             s_hbm.at[pl.ds(nbase + soff, sz)], sem_s)

        pltpu.make_async_copy(acc_sh.at[pl.ds(abase, npw)],
                              n0s_hbm.at[pl.ds(nbase, npw)], sem_o).wait()
        zero_slab()
        for b, (soff, sz) in enumerate(s_parts):
            pltpu.make_async_copy(rows_v.at[b, pl.ds(0, sz)],
                                  s_hbm.at[pl.ds(nbase + soff, sz)],
                                  sem_s).wait()
        one_table(n1_hbm, n1s_hbm)
        pltpu.make_async_copy(acc_sh.at[pl.ds(abase, npw)],
                              n1s_hbm.at[pl.ds(nbase, npw)], sem_o).wait()

    return sc_kernel(features, nodes_p, n0_p, n1_p)


def _tc_matmul(weight, s, n0s, n1s):
    bp = s.shape[0]
    blk = 512
    dn = (((1,), (1,)), ((), ()))

    def body(w_ref, s_ref, n0_ref, n1_ref, o_ref):
        w = w_ref[...]
        acc = lax.dot_general(w[:, 0:D], s_ref[...], dn,
                              preferred_element_type=jnp.float32)
        wn = w[:, D:3 * D] * jnp.float32(1.0 / K)
        acc = acc + lax.dot_general(wn[:, 0:D], n0_ref[...], dn,
                                    preferred_element_type=jnp.float32)
        acc = acc + lax.dot_general(wn[:, D:2 * D], n1_ref[...], dn,
                                    preferred_element_type=jnp.float32)
        o_ref[...] = jnp.maximum(acc, 0.0)

    return pl.pallas_call(
        body,
        grid=(bp // blk,),
        in_specs=[
            pl.BlockSpec((D, 3 * D), lambda i: (0, 0)),
            pl.BlockSpec((blk, D), lambda i: (i, 0)),
            pl.BlockSpec((blk, D), lambda i: (i, 0)),
            pl.BlockSpec((blk, D), lambda i: (i, 0)),
        ],
        out_specs=pl.BlockSpec((D, blk), lambda i: (0, i)),
        out_shape=jax.ShapeDtypeStruct((D, bp), jnp.float32),
    )(weight, s, n0s, n1s)


def kernel(nodes, neigh0, neigh1, features, weight):
    b = nodes.shape[0]
    bp = -(-b // (SCH * NW)) * (SCH * NW)
    pad = bp - b
    nodes_p = jnp.concatenate(
        [nodes.astype(jnp.int32), jnp.zeros((pad,), jnp.int32)])
    n0_p = jnp.concatenate(
        [neigh0.astype(jnp.int32).reshape(-1), jnp.zeros((pad * K,), jnp.int32)])
    n1_p = jnp.concatenate(
        [neigh1.astype(jnp.int32).reshape(-1), jnp.zeros((pad * K,), jnp.int32)])
    s, n0s, n1s = _sc_gather_sum(features, nodes_p, n0_p, n1_p, bp)
    out = _tc_matmul(weight, s, n0s, n1s)
    return out[:, :b]


# speedup vs baseline: 3.4085x; 3.4085x over previous
import functools

import jax
import jax.numpy as jnp
from jax import lax
from jax.experimental import pallas as pl
from jax.experimental.pallas import tpu as pltpu
from jax.experimental.pallas import tpu_sc as plsc

D = 128
K = 16
NC = 2
NS = 16
NW = NC * NS
CH = 8
CHK = CH * K
SCH = 64
ZR = 32
NB = 4


def _sc_gather_sum(features, nodes_p, n0_p, n1_p, b, bpo):
    npw = -(-b // (SCH * NW)) * SCH
    assert b % 8 == 0 and b >= npw
    last_base = b - npw
    n_chunks = npw // CH
    s_chunks = npw // SCH
    z_copies = npw // ZR
    mesh = plsc.VectorSubcoreMesh(core_axis_name="c", subcore_axis_name="s")
    f32 = jnp.float32

    @functools.partial(
        pl.kernel,
        out_type=(
            jax.ShapeDtypeStruct((bpo, D), f32),
            jax.ShapeDtypeStruct((bpo, D), f32),
            jax.ShapeDtypeStruct((bpo, D), f32),
        ),
        mesh=mesh,
        scratch_types=[
            pltpu.VMEM((npw,), jnp.int32),
            pltpu.VMEM((npw * K,), jnp.int32),
            pltpu.VMEM((NB, CHK, D), f32),
            pltpu.VMEM((ZR, D), f32),
            pltpu.VMEM((NB, CHK), jnp.int32),
            pltpu.VMEM_SHARED((NS * npw, D), f32),
            pltpu.SemaphoreType.DMA,
            pltpu.SemaphoreType.DMA,
            pltpu.SemaphoreType.DMA,
            pltpu.SemaphoreType.DMA,
            pltpu.SemaphoreType.DMA,
            pltpu.SemaphoreType.DMA,
            pltpu.SemaphoreType.DMA,
            pltpu.SemaphoreType.DMA,
            pltpu.SemaphoreType.DMA,
            pltpu.SemaphoreType.DMA,
        ],
    )
    def sc_kernel(feat_hbm, nodes_hbm, n0_hbm, n1_hbm,
                  s_hbm, n0s_hbm, n1s_hbm,
                  sidx_v, nidx_v, rows_v, zer_v, aidx_v, acc_sh,
                  sg0, sg1, sg2, sg3, sa0, sa1, sa2, sa3, sem_s, sem_o):
        sem_g = (sg0, sg1, sg2, sg3)
        sem_a = (sa0, sa1, sa2, sa3)
        sid = lax.axis_index("s")
        wid = sid * NC + lax.axis_index("c")
        nbase = jnp.minimum(wid * npw, last_base)
        abase = sid * npw

        for r in range(ZR):
            for j in range(D // 16):
                zer_v[r, pl.ds(j * 16, 16)] = jnp.zeros((16,), f32)

        def zero_slab():
            for zi in range(z_copies):
                pltpu.sync_copy(zer_v, acc_sh.at[pl.ds(abase + zi * ZR, ZR)])

        zero_slab()

        pltpu.sync_copy(nodes_hbm.at[pl.ds(nbase, npw)], sidx_v)

        def one_table(tbl_hbm, out_hbm):
            ibase = nbase * K
            pltpu.sync_copy(tbl_hbm.at[pl.ds(ibase, npw * K)], nidx_v)

            def issue_gather(ci, b):
                pltpu.async_copy(
                    feat_hbm.at[nidx_v.at[pl.ds(ci * CHK, CHK)]],
                    rows_v.at[b], sem_g[b])

            def wait_gather(b):
                pltpu.make_async_copy(
                    feat_hbm.at[nidx_v.at[pl.ds(0, CHK)]],
                    rows_v.at[b], sem_g[b]).wait()

            def issue_add(ci, b):
                for j in range(CH):
                    aidx_v[b, pl.ds(j * K, K)] = lax.full(
                        (16,), abase + ci * CH + j, jnp.int32)
                pltpu.async_copy(rows_v.at[b], acc_sh.at[aidx_v.at[b]],
                                 sem_a[b], add=True)

            def wait_add(b):
                pltpu.make_async_copy(rows_v.at[b], acc_sh.at[aidx_v.at[b]],
                                      sem_a[b]).wait()

            for b in range(NB):
                issue_gather(b, b)

            @pl.loop(0, n_chunks, step=NB)
            def _(g):
                for b in range(NB):
                    ci = g + b
                    pb = (b - 1) % NB
                    pci = ci - 1

                    @pl.when((pci >= 0) & (pci + NB < n_chunks))
                    def _():
                        wait_add(pb)
                        issue_gather(pci + NB, pb)

                    wait_gather(b)
                    issue_add(ci, b)

            for b in range(NB):
                wait_add(b)
            pltpu.async_copy(acc_sh.at[pl.ds(abase, npw)],
                             out_hbm.at[pl.ds(nbase, npw)], sem_o)

        one_table(n0_hbm, n0s_hbm)

        s_parts = []
        off = 0
        while off < npw:
            sz = min(CHK, npw - off)
            s_parts.append((off, sz))
            off += sz
        for b, (soff, sz) in enumerate(s_parts):
            pltpu.async_copy(feat_hbm.at[sidx_v.at[pl.ds(soff, sz)]],
                             rows_v.at[b, pl.ds(0, sz)], sem_g[b])
        for b, (soff, sz) in enumerate(s_parts):
            pltpu.make_async_copy(feat_hbm.at[sidx_v.at[pl.ds(soff, sz)]],
                                  rows_v.at[b, pl.ds(0, sz)], sem_g[b]).wait()
            pltpu.async_copy(rows_v.at[b, pl.ds(0, sz)],
                             s_hbm.at[pl.ds(nbase + soff, sz)], sem_s)

        pltpu.make_async_copy(acc_sh.at[pl.ds(abase, npw)],
                              n0s_hbm.at[pl.ds(nbase, npw)], sem_o).wait()
        zero_slab()
        for b, (soff, sz) in enumerate(s_parts):
            pltpu.make_async_copy(rows_v.at[b, pl.ds(0, sz)],
                                  s_hbm.at[pl.ds(nbase + soff, sz)],
                                  sem_s).wait()
        one_table(n1_hbm, n1s_hbm)
        pltpu.make_async_copy(acc_sh.at[pl.ds(abase, npw)],
                              n1s_hbm.at[pl.ds(nbase, npw)], sem_o).wait()

    return sc_kernel(features, nodes_p, n0_p, n1_p)


def _tc_matmul(weight, s, n0s, n1s):
    bp = s.shape[0]
    blk = 512
    dn = (((1,), (1,)), ((), ()))

    def body(w_ref, s_ref, n0_ref, n1_ref, o_ref):
        w = w_ref[...]
        acc = lax.dot_general(w[:, 0:D], s_ref[...], dn,
                              preferred_element_type=jnp.float32)
        wn = w[:, D:3 * D] * jnp.float32(1.0 / K)
        acc = acc + lax.dot_general(wn[:, 0:D], n0_ref[...], dn,
                                    preferred_element_type=jnp.float32)
        acc = acc + lax.dot_general(wn[:, D:2 * D], n1_ref[...], dn,
                                    preferred_element_type=jnp.float32)
        o_ref[...] = jnp.maximum(acc, 0.0)

    return pl.pallas_call(
        body,
        grid=(bp // blk,),
        in_specs=[
            pl.BlockSpec((D, 3 * D), lambda i: (0, 0)),
            pl.BlockSpec((blk, D), lambda i: (i, 0)),
            pl.BlockSpec((blk, D), lambda i: (i, 0)),
            pl.BlockSpec((blk, D), lambda i: (i, 0)),
        ],
        out_specs=pl.BlockSpec((D, blk), lambda i: (0, i)),
        out_shape=jax.ShapeDtypeStruct((D, bp), jnp.float32),
    )(weight, s, n0s, n1s)


def kernel(nodes, neigh0, neigh1, features, weight):
    b = nodes.shape[0]
    bpo = -(-b // 512) * 512
    s, n0s, n1s = _sc_gather_sum(
        features, nodes.astype(jnp.int32),
        neigh0.astype(jnp.int32).reshape(-1),
        neigh1.astype(jnp.int32).reshape(-1), b, bpo)
    out = _tc_matmul(weight, s, n0s, n1s)
    return out[:, :b]
